# baseline (speedup 1.0000x reference)
import functools

import jax
import jax.numpy as jnp
from jax import lax
from jax.experimental import pallas as pl
from jax.experimental.pallas import tpu as pltpu
from jax.experimental.pallas import tpu_sc as plsc

B = 16384
C = 1000
FILL = 1.0 / (C - 1)

NC = 2
NS = 16
NW = NC * NS
RPW = B // NW
CH = 32
NCHUNK = RPW // CH
L = 16


@functools.partial(
    pl.kernel,
    mesh=plsc.VectorSubcoreMesh(core_axis_name="c", subcore_axis_name="s"),
    out_type=jax.ShapeDtypeStruct((B * C,), jnp.float32),
    scratch_types=[
        pltpu.VMEM((CH * C,), jnp.float32),
        pltpu.VMEM((RPW,), jnp.int32),
        pltpu.VMEM((RPW // 128, 128), jnp.int32),
        pltpu.VMEM((128,), jnp.float32),
        pltpu.SemaphoreType.DMA,
        pltpu.SemaphoreType.DMA,
    ],
)
def _virtual_teacher(y_hbm, out_hbm, buf, yv, idx2, zv, sem_fill, sem_sc):
    wid = lax.axis_index("s") * NC + lax.axis_index("c")
    base = wid * RPW

    pltpu.sync_copy(y_hbm.at[pl.ds(base, RPW)], yv)

    fill_vec = jnp.full((L,), FILL, dtype=jnp.float32)

    def fill_body(i, carry):
        buf[pl.ds(i * L, L)] = fill_vec
        return carry

    lax.fori_loop(0, CH * C // L, fill_body, 0, unroll=8)

    zvec = jnp.zeros((L,), jnp.float32)
    for k in range(128 // L):
        zv[pl.ds(k * L, L)] = zvec

    fills = [
        pltpu.async_copy(
            buf, out_hbm.at[pl.ds((base + t * CH) * C, CH * C)], sem_fill
        )
        for t in range(NCHUNK)
    ]

    iota = lax.iota(jnp.int32, L)
    for k in range(RPW // L):
        yvals = yv[pl.ds(k * L, L)]
        flat = ((base + k * L) + iota) * C + yvals
        idx2[k // 8, pl.ds((k % 8) * L, L)] = flat

    for cp in fills:
        cp.wait()

    scats = [
        pltpu.async_copy(zv, out_hbm.at[idx2.at[j]], sem_sc)
        for j in range(RPW // 128)
    ]
    for cp in scats:
        cp.wait()


def kernel(x, y):
    del x
    out = _virtual_teacher(y.astype(jnp.int32))
    return out.reshape(B, C)


# speedup vs baseline: 1.7319x; 1.7319x over previous
import functools

import jax
import jax.numpy as jnp
from jax import lax
from jax.experimental import pallas as pl
from jax.experimental.pallas import tpu as pltpu
from jax.experimental.pallas import tpu_sc as plsc

B = 16384
C = 1000
FILL = 1.0 / (C - 1)

NC = 2
NS = 16
NW = NC * NS
RPW = B // NW
CH = 32
NCHUNK = RPW // CH
L = 16


@functools.partial(
    pl.kernel,
    mesh=plsc.VectorSubcoreMesh(core_axis_name="c", subcore_axis_name="s"),
    out_type=jax.ShapeDtypeStruct((B, C), jnp.float32),
    scratch_types=[
        pltpu.VMEM((CH, C), jnp.float32),
        pltpu.VMEM((CH, C), jnp.float32),
        pltpu.VMEM((RPW,), jnp.int32),
        pltpu.SemaphoreType.DMA,
        pltpu.SemaphoreType.DMA,
    ],
)
def _virtual_teacher(y_hbm, out_hbm, buf0, buf1, yv, sem0, sem1):
    wid = lax.axis_index("s") * NC + lax.axis_index("c")
    base = wid * RPW

    pltpu.sync_copy(y_hbm.at[pl.ds(base, RPW)], yv)

    fill_vec = jnp.full((L,), FILL, dtype=jnp.float32)
    iota = lax.iota(jnp.int32, L)

    def fill_row(r, carry):
        def col_body(k, c2):
            start = pl.multiple_of(k * L, L)
            buf0[r, pl.ds(start, L)] = fill_vec
            buf1[r, pl.ds(start, L)] = fill_vec
            return c2

        lax.fori_loop(0, C // L + 1, col_body, 0, unroll=8)
        return carry

    lax.fori_loop(0, CH, fill_row, 0)

    bufs = (buf0, buf1)
    sems = (sem0, sem1)

    def chunk_rewrite(t, buf, zero):
        for k in range(CH // L):
            ys = yv[pl.ds(t * CH + k * L, L)]
            for j in range(L):
                y_r = ys[j]
                start = pl.multiple_of((y_r // L) * L, L)
                if zero:
                    vec = jnp.where(iota == (y_r & (L - 1)), 0.0, FILL).astype(
                        jnp.float32
                    )
                else:
                    vec = fill_vec
                buf[k * L + j, pl.ds(start, L)] = vec

    def fire(t, b):
        return pltpu.async_copy(
            bufs[b], out_hbm.at[pl.ds(base + t * CH, CH)], sems[b]
        )

    def drain_one(b):
        pltpu.make_async_copy(
            bufs[b], out_hbm.at[pl.ds(base, CH)], sems[b]
        ).wait()

    chunk_rewrite(0, buf0, zero=True)
    fire(0, 0)
    chunk_rewrite(1, buf1, zero=True)
    fire(1, 1)

    def pair_body(g, carry):
        t0 = 2 * g
        for b in (0, 1):
            t = t0 + b
            drain_one(b)
            chunk_rewrite(t - 2, bufs[b], zero=False)
            chunk_rewrite(t, bufs[b], zero=True)
            fire(t, b)
        return carry

    lax.fori_loop(1, NCHUNK // 2, pair_body, 0)

    drain_one(0)
    drain_one(1)


def kernel(x, y):
    del x
    return _virtual_teacher(y.astype(jnp.int32))


# speedup vs baseline: 3.5213x; 2.0332x over previous
import functools

import jax
import jax.numpy as jnp
from jax import lax
from jax.experimental import pallas as pl
from jax.experimental.pallas import tpu as pltpu
from jax.experimental.pallas import tpu_sc as plsc

B = 16384
C = 1000
FILL = 1.0 / (C - 1)

NC = 2
NS = 16
NW = NC * NS
CPW = B // NW
CB = 128
NJ = CPW // CB
HA = 496
HB = C - HA
L = 16


@functools.partial(
    pl.kernel,
    mesh=plsc.VectorSubcoreMesh(core_axis_name="c", subcore_axis_name="s"),
    out_type=jax.ShapeDtypeStruct((C, B), jnp.float32),
    scratch_types=[
        pltpu.VMEM((HA, CB), jnp.float32),
        pltpu.VMEM((HB, CB), jnp.float32),
        pltpu.VMEM((CPW,), jnp.int32),
        pltpu.SemaphoreType.DMA,
        pltpu.SemaphoreType.DMA,
    ],
)
def _virtual_teacher(y_hbm, out_hbm, buf_a, buf_b, yv, sem_a, sem_b):
    wid = lax.axis_index("s") * NC + lax.axis_index("c")
    base = wid * CPW

    pltpu.sync_copy(y_hbm.at[pl.ds(base, CPW)], yv)

    fill_vec = jnp.full((L,), FILL, dtype=jnp.float32)
    iota = lax.iota(jnp.int32, L)

    def fill_a(r, carry):
        for k in range(CB // L):
            buf_a[r, pl.ds(k * L, L)] = fill_vec
        return carry

    def fill_b(r, carry):
        for k in range(CB // L):
            buf_b[r, pl.ds(k * L, L)] = fill_vec
        return carry

    lax.fori_loop(0, HA, fill_a, 0)
    lax.fori_loop(0, HB, fill_b, 0)

    def scan_pass(buf, h0, hrows, j, value):
        def group(g, carry):
            ys = yv[pl.ds(j * CB + g * L, L)]
            cstart = pl.multiple_of(g * L, L)
            for jj in range(L):
                y_r = ys[jj]
                hit = jnp.logical_and(y_r >= h0, y_r < h0 + hrows)
                row = jnp.clip(y_r - h0, 0, hrows - 1)

                @pl.when(hit)
                def _():
                    old = buf[row, pl.ds(cstart, L)]
                    buf[row, pl.ds(cstart, L)] = jnp.where(
                        iota == jj, value, old
                    )

            return carry

        lax.fori_loop(0, CB // L, group, 0)

    def fire(buf, h0, j, sem):
        return pltpu.async_copy(
            buf,
            out_hbm.at[pl.ds(h0, buf.shape[0]), pl.ds(base + j * CB, CB)],
            sem,
        )

    def drain(buf, h0, sem):
        pltpu.make_async_copy(
            buf, out_hbm.at[pl.ds(h0, buf.shape[0]), pl.ds(base, CB)], sem
        ).wait()

    scan_pass(buf_a, 0, HA, 0, 0.0)
    fire(buf_a, 0, 0, sem_a)
    scan_pass(buf_b, HA, HB, 0, 0.0)
    fire(buf_b, HA, 0, sem_b)

    def chunk_body(j, carry):
        drain(buf_a, 0, sem_a)
        scan_pass(buf_a, 0, HA, j - 1, FILL)
        scan_pass(buf_a, 0, HA, j, 0.0)
        fire(buf_a, 0, j, sem_a)
        drain(buf_b, HA, sem_b)
        scan_pass(buf_b, HA, HB, j - 1, FILL)
        scan_pass(buf_b, HA, HB, j, 0.0)
        fire(buf_b, HA, j, sem_b)
        return carry

    lax.fori_loop(1, NJ, chunk_body, 0)

    drain(buf_a, 0, sem_a)
    drain(buf_b, HA, sem_b)


def kernel(x, y):
    del x
    return _virtual_teacher(y.astype(jnp.int32)).T


# speedup vs baseline: 3.6917x; 1.0484x over previous
import functools

import jax
import jax.numpy as jnp
from jax import lax
from jax.experimental import pallas as pl
from jax.experimental.pallas import tpu as pltpu
from jax.experimental.pallas import tpu_sc as plsc

B = 16384
C = 1000
FILL = 1.0 / (C - 1)

NC = 2
NS = 16
NW = NC * NS
CPW = B // NW
CB = 128
NJ = CPW // CB
HA = 496
HB = C - HA
L = 16


@functools.partial(
    pl.kernel,
    mesh=plsc.VectorSubcoreMesh(core_axis_name="c", subcore_axis_name="s"),
    out_type=jax.ShapeDtypeStruct((C, B), jnp.float32),
    scratch_types=[
        pltpu.VMEM((HA, CB), jnp.float32),
        pltpu.VMEM((HB, CB), jnp.float32),
        pltpu.VMEM((CPW,), jnp.int32),
        pltpu.SemaphoreType.DMA,
        pltpu.SemaphoreType.DMA,
    ],
)
def _virtual_teacher(y_hbm, out_hbm, buf_a, buf_b, yv, sem_a, sem_b):
    wid = lax.axis_index("s") * NC + lax.axis_index("c")
    base = wid * CPW

    pltpu.sync_copy(y_hbm.at[pl.ds(base, CPW)], yv)

    fill_vec = jnp.full((L,), FILL, dtype=jnp.float32)
    iota = lax.iota(jnp.int32, L)

    def fill_a(r, carry):
        for k in range(CB // L):
            buf_a[r, pl.ds(k * L, L)] = fill_vec
        return carry

    def fill_b(r, carry):
        for k in range(CB // L):
            buf_b[r, pl.ds(k * L, L)] = fill_vec
        return carry

    def scan_pass(buf, h0, hrows, j, value):
        def group(g, carry):
            ys = yv[pl.ds(j * CB + g * L, L)]
            cstart = pl.multiple_of(g * L, L)
            for jj in range(L):
                y_r = ys[jj]
                hit = jnp.logical_and(y_r >= h0, y_r < h0 + hrows)
                row = jnp.clip(y_r - h0, 0, hrows - 1)
                lane = jnp.where(hit, jj, -1)
                old = buf[row, pl.ds(cstart, L)]
                buf[row, pl.ds(cstart, L)] = jnp.where(iota == lane, value, old)
            return carry

        lax.fori_loop(0, CB // L, group, 0)

    def fire(buf, h0, j, sem):
        return pltpu.async_copy(
            buf,
            out_hbm.at[pl.ds(h0, buf.shape[0]), pl.ds(base + j * CB, CB)],
            sem,
        )

    def drain(buf, h0, sem):
        pltpu.make_async_copy(
            buf, out_hbm.at[pl.ds(h0, buf.shape[0]), pl.ds(base, CB)], sem
        ).wait()

    lax.fori_loop(0, HA, fill_a, 0)
    scan_pass(buf_a, 0, HA, 0, 0.0)
    fire(buf_a, 0, 0, sem_a)
    lax.fori_loop(0, HB, fill_b, 0)
    scan_pass(buf_b, HA, HB, 0, 0.0)
    fire(buf_b, HA, 0, sem_b)

    def chunk_body(j, carry):
        drain(buf_a, 0, sem_a)
        scan_pass(buf_a, 0, HA, j - 1, FILL)
        scan_pass(buf_a, 0, HA, j, 0.0)
        fire(buf_a, 0, j, sem_a)
        drain(buf_b, HA, sem_b)
        scan_pass(buf_b, HA, HB, j - 1, FILL)
        scan_pass(buf_b, HA, HB, j, 0.0)
        fire(buf_b, HA, j, sem_b)
        return carry

    lax.fori_loop(1, NJ, chunk_body, 0)

    drain(buf_a, 0, sem_a)
    drain(buf_b, HA, sem_b)


def kernel(x, y):
    del x
    return _virtual_teacher(y.astype(jnp.int32)).T
